# baseline (speedup 1.0000x reference)
import functools

import jax
import jax.numpy as jnp
from jax import lax
from jax.experimental import pallas as pl
from jax.experimental.pallas import tpu as pltpu
from jax.experimental.pallas import tpu_sc as plsc

B = 1024
S = 50
OUT_W = 1100
L = 16
NC, NS = 2, 16
NW = NC * NS
RPW = B // NW
GROUPS = RPW // L
HWORDS = RPW * OUT_W

_mesh = plsc.VectorSubcoreMesh(core_axis_name="c", subcore_axis_name="s")


@functools.partial(
    pl.kernel,
    out_type=jax.ShapeDtypeStruct((B * OUT_W,), jnp.float32),
    mesh=_mesh,
    scratch_types=[
        pltpu.VMEM((RPW * S,), jnp.int32),
        pltpu.VMEM((HWORDS,), jnp.float32),
    ],
    compiler_params=pltpu.CompilerParams(
        use_tc_tiling_on_sc=False, needs_layout_passes=False
    ),
)
def _bow_kernel(idx_hbm, out_hbm, idx_v, hist_v):
    wid = lax.axis_index("s") * NC + lax.axis_index("c")
    base = wid * RPW

    pltpu.sync_copy(idx_hbm.at[pl.ds(base * S, RPW * S)], idx_v)

    zeros = jnp.zeros((L,), jnp.float32)

    def _zero(i, carry):
        hist_v[pl.ds(i * L, L)] = zeros
        return carry

    lax.fori_loop(0, HWORDS // L, _zero, 0)

    lane_rows = lax.iota(jnp.int32, L)
    ones = jnp.ones((L,), jnp.float32)

    def _group(g, carry):
        rows = lane_rows + g * L
        row_off = rows * OUT_W
        idx_off = rows * S

        def _step(t, carry2):
            tok = plsc.load_gather(idx_v, [idx_off + t])
            dst = row_off + jnp.maximum(tok - 1, 0)
            plsc.addupdate_scatter(hist_v, [dst], ones, mask=tok > 0)
            return carry2

        return lax.fori_loop(0, S, _step, carry)

    lax.fori_loop(0, GROUPS, _group, 0)

    pltpu.sync_copy(hist_v, out_hbm.at[pl.ds(base * OUT_W, HWORDS)])


def kernel(inputs):
    return _bow_kernel(inputs.reshape(B * S)).reshape(B, OUT_W)


# speedup vs baseline: 1.2352x; 1.2352x over previous
import functools

import jax
import jax.numpy as jnp
from jax import lax
from jax.experimental import pallas as pl
from jax.experimental.pallas import tpu as pltpu
from jax.experimental.pallas import tpu_sc as plsc

B = 1024
S = 50
OUT_W = 1100
L = 16
NC, NS = 2, 16
NW = NC * NS
RPW = B // NW
GROUPS = RPW // L
HWORDS = RPW * OUT_W

_mesh = plsc.VectorSubcoreMesh(core_axis_name="c", subcore_axis_name="s")


@functools.partial(
    pl.kernel,
    out_type=jax.ShapeDtypeStruct((B * OUT_W,), jnp.float32),
    mesh=_mesh,
    scratch_types=[
        pltpu.VMEM((RPW * S,), jnp.int32),
        pltpu.VMEM((HWORDS,), jnp.float32),
        pltpu.SemaphoreType.DMA,
    ],
    compiler_params=pltpu.CompilerParams(
        use_tc_tiling_on_sc=False, needs_layout_passes=False
    ),
)
def _bow_kernel(idx_hbm, out_hbm, idx_v, hist_v, in_sem):
    wid = lax.axis_index("s") * NC + lax.axis_index("c")
    base = wid * RPW

    in_dma = pltpu.async_copy(
        idx_hbm.at[pl.ds(base * S, RPW * S)], idx_v, in_sem
    )

    zeros = jnp.zeros((L,), jnp.float32)

    @plsc.parallel_loop(0, HWORDS, step=L, unroll=16)
    def _zero(i):
        hist_v[pl.ds(i, L)] = zeros

    in_dma.wait()

    lane_rows = lax.iota(jnp.int32, L)
    ones = jnp.ones((L,), jnp.float32)
    for g in range(GROUPS):
        rows = lane_rows + g * L
        row_off = rows * OUT_W
        idx_off = rows * S
        for t in range(S):
            tok = plsc.load_gather(idx_v, [idx_off + t])
            dst = row_off + jnp.maximum(tok - 1, 0)
            plsc.addupdate_scatter(hist_v, [dst], ones, mask=tok > 0)

    pltpu.sync_copy(hist_v, out_hbm.at[pl.ds(base * OUT_W, HWORDS)])


def kernel(inputs):
    return _bow_kernel(inputs.reshape(B * S)).reshape(B, OUT_W)


# speedup vs baseline: 1.3118x; 1.0620x over previous
import functools

import jax
import jax.numpy as jnp
from jax import lax
from jax.experimental import pallas as pl
from jax.experimental.pallas import tpu as pltpu
from jax.experimental.pallas import tpu_sc as plsc

B = 1024
S = 50
OUT_W = 1100
L = 16
NC, NS = 2, 16
NW = NC * NS
RPW = B // NW
GROUPS = RPW // L
ZFULL = (OUT_W // L) * L

_mesh = plsc.VectorSubcoreMesh(core_axis_name="c", subcore_axis_name="s")


@functools.partial(
    pl.kernel,
    out_type=jax.ShapeDtypeStruct((B, OUT_W), jnp.float32),
    mesh=_mesh,
    scratch_types=[
        pltpu.VMEM((RPW, S), jnp.int32),
        pltpu.VMEM((RPW, OUT_W), jnp.float32),
        pltpu.SemaphoreType.DMA,
    ],
    compiler_params=pltpu.CompilerParams(
        use_tc_tiling_on_sc=False, needs_layout_passes=False
    ),
)
def _bow_kernel(idx_hbm, out_hbm, idx_v, hist_v, in_sem):
    wid = lax.axis_index("s") * NC + lax.axis_index("c")
    base = wid * RPW

    in_dma = pltpu.async_copy(idx_hbm.at[pl.ds(base, RPW), :], idx_v, in_sem)

    zeros = jnp.zeros((L,), jnp.float32)
    lanes = lax.iota(jnp.int32, L)
    tail_cols = lanes + ZFULL
    tail_mask = lanes < (OUT_W - ZFULL)

    def _zero_row(r, carry):
        @plsc.parallel_loop(0, ZFULL, step=L, unroll=8)
        def _zero(i):
            hist_v[r, pl.ds(i, L)] = zeros

        plsc.store_scatter(
            hist_v, [jnp.full((L,), r, jnp.int32), tail_cols], zeros,
            mask=tail_mask,
        )
        return carry

    lax.fori_loop(0, RPW, _zero_row, 0)

    in_dma.wait()

    ones = jnp.ones((L,), jnp.float32)
    for g in range(GROUPS):
        rows = lanes + g * L

        @plsc.parallel_loop(0, S, step=1, unroll=5)
        def _step(t):
            tok = plsc.load_gather(idx_v, [rows, jnp.full((L,), t, jnp.int32)])
            dst = jnp.maximum(tok - 1, 0)
            plsc.addupdate_scatter(hist_v, [rows, dst], ones, mask=tok > 0)

    pltpu.sync_copy(hist_v, out_hbm.at[pl.ds(base, RPW), :])


def kernel(inputs):
    return _bow_kernel(inputs)


# speedup vs baseline: 1.4735x; 1.1233x over previous
import functools

import jax
import jax.numpy as jnp
from jax import lax
from jax.experimental import pallas as pl
from jax.experimental.pallas import tpu as pltpu
from jax.experimental.pallas import tpu_sc as plsc

B = 1024
S = 50
OUT_W = 1100
L = 16
NC, NS = 2, 16
NW = NC * NS
CPW = B // NW
GROUPS = CPW // L

_mesh = plsc.VectorSubcoreMesh(core_axis_name="c", subcore_axis_name="s")


@functools.partial(
    pl.kernel,
    out_type=jax.ShapeDtypeStruct((OUT_W, B), jnp.float32),
    mesh=_mesh,
    scratch_types=[
        pltpu.VMEM((S, CPW), jnp.int32),
        pltpu.VMEM((OUT_W, CPW), jnp.float32),
        pltpu.SemaphoreType.DMA,
    ],
    compiler_params=pltpu.CompilerParams(
        use_tc_tiling_on_sc=False, needs_layout_passes=False
    ),
)
def _bow_kernel(idx_hbm, out_hbm, idx_v, hist_v, in_sem):
    wid = lax.axis_index("s") * NC + lax.axis_index("c")
    base = wid * CPW

    in_dma = pltpu.async_copy(idx_hbm.at[:, pl.ds(base, CPW)], idx_v, in_sem)

    zeros = jnp.zeros((L,), jnp.float32)

    @plsc.parallel_loop(0, OUT_W, step=1, unroll=8)
    def _zero(r):
        hist_v[r, pl.ds(0, L)] = zeros
        hist_v[r, pl.ds(L, L)] = zeros

    in_dma.wait()

    ones = jnp.ones((L,), jnp.float32)
    lanes = lax.iota(jnp.int32, L)
    for g in range(GROUPS):
        cols = lanes + g * L

        @plsc.parallel_loop(0, S, step=1, unroll=5)
        def _step(t):
            tok = idx_v[t, pl.ds(g * L, L)]
            dst = jnp.maximum(tok - 1, 0)
            plsc.addupdate_scatter(hist_v, [dst, cols], ones, mask=tok > 0)

    pltpu.sync_copy(hist_v, out_hbm.at[:, pl.ds(base, CPW)])


def kernel(inputs):
    return _bow_kernel(inputs.T).T


# speedup vs baseline: 1.4770x; 1.0024x over previous
import functools

import jax
import jax.numpy as jnp
from jax import lax
from jax.experimental import pallas as pl
from jax.experimental.pallas import tpu as pltpu
from jax.experimental.pallas import tpu_sc as plsc

B = 1024
S = 50
OUT_W = 1100
L = 16
NC, NS = 2, 16
NW = NC * NS
CPW = B // NW
GROUPS = CPW // L

_mesh = plsc.VectorSubcoreMesh(core_axis_name="c", subcore_axis_name="s")


@functools.partial(
    pl.kernel,
    out_type=jax.ShapeDtypeStruct((OUT_W, B), jnp.float32),
    mesh=_mesh,
    scratch_types=[
        pltpu.VMEM((S, CPW), jnp.int32),
        pltpu.VMEM((OUT_W, CPW), jnp.float32),
        pltpu.SemaphoreType.DMA,
    ],
    compiler_params=pltpu.CompilerParams(
        use_tc_tiling_on_sc=False,
        needs_layout_passes=False,
        skip_device_barrier=True,
    ),
)
def _bow_kernel(idx_hbm, out_hbm, idx_v, hist_v, in_sem):
    wid = lax.axis_index("s") * NC + lax.axis_index("c")
    base = wid * CPW

    in_dma = pltpu.async_copy(idx_hbm.at[:, pl.ds(base, CPW)], idx_v, in_sem)

    zeros = jnp.zeros((L,), jnp.float32)

    @plsc.parallel_loop(0, OUT_W, step=1, unroll=8)
    def _zero(r):
        hist_v[r, pl.ds(0, L)] = zeros
        hist_v[r, pl.ds(L, L)] = zeros

    in_dma.wait()

    ones = jnp.ones((L,), jnp.float32)
    lanes = lax.iota(jnp.int32, L)
    for g in range(GROUPS):
        cols = lanes + g * L

        @plsc.parallel_loop(0, S, step=1, unroll=5)
        def _step(t):
            tok = idx_v[t, pl.ds(g * L, L)]
            dst = jnp.maximum(tok - 1, 0)
            plsc.addupdate_scatter(hist_v, [dst, cols], ones, mask=tok > 0)

    pltpu.sync_copy(hist_v, out_hbm.at[:, pl.ds(base, CPW)])


def kernel(inputs):
    return _bow_kernel(inputs.T).T


# speedup vs baseline: 1.5179x; 1.0276x over previous
import functools

import jax
import jax.numpy as jnp
from jax import lax
from jax.experimental import pallas as pl
from jax.experimental.pallas import tpu as pltpu
from jax.experimental.pallas import tpu_sc as plsc

B = 1024
S = 50
OUT_W = 1100
RT = 138
CT = B // 128
L = 16
NC, NS = 2, 16
NW = NC * NS
CPW = B // NW
GROUPS = CPW // L

_mesh = plsc.VectorSubcoreMesh(core_axis_name="c", subcore_axis_name="s")


@functools.partial(
    pl.kernel,
    out_type=jax.ShapeDtypeStruct((RT, CT, 8, 128), jnp.float32),
    mesh=_mesh,
    scratch_types=[
        pltpu.VMEM((S, CPW), jnp.int32),
        pltpu.VMEM((RT, 8, CPW), jnp.float32),
        pltpu.SemaphoreType.DMA,
    ],
    compiler_params=pltpu.CompilerParams(
        use_tc_tiling_on_sc=False, needs_layout_passes=False
    ),
)
def _bow_kernel(idx_hbm, out_hbm, idx_v, hist_v, in_sem):
    wid = lax.axis_index("s") * NC + lax.axis_index("c")
    base = wid * CPW
    ct = base // 128
    ci0 = base % 128

    in_dma = pltpu.async_copy(idx_hbm.at[:, pl.ds(base, CPW)], idx_v, in_sem)

    zeros = jnp.zeros((L,), jnp.float32)

    @plsc.parallel_loop(0, RT * 8, step=1, unroll=8)
    def _zero(r):
        rt = r >> 3
        ri = r & 7
        hist_v[rt, ri, pl.ds(0, L)] = zeros
        hist_v[rt, ri, pl.ds(L, L)] = zeros

    in_dma.wait()

    ones = jnp.ones((L,), jnp.float32)
    lanes = lax.iota(jnp.int32, L)
    for g in range(GROUPS):
        cols = lanes + g * L

        @plsc.parallel_loop(0, S, step=1, unroll=5)
        def _step(t):
            tok = idx_v[t, pl.ds(g * L, L)]
            r = jnp.maximum(tok - 1, 0)
            plsc.addupdate_scatter(
                hist_v, [r >> 3, r & 7, cols], ones, mask=tok > 0
            )

    pltpu.sync_copy(hist_v, out_hbm.at[:, ct, :, pl.ds(ci0, CPW)])


def kernel(inputs):
    out4 = _bow_kernel(inputs.T)
    x = out4.transpose(0, 2, 1, 3).reshape(RT * 8, B)
    return x[:OUT_W].T


# speedup vs baseline: 1.8540x; 1.2215x over previous
import functools

import jax
import jax.numpy as jnp
from jax import lax
from jax.experimental import pallas as pl
from jax.experimental.pallas import tpu as pltpu
from jax.experimental.pallas import tpu_sc as plsc

B = 1024
S = 50
OUT_W = 1100
RT = 138
CT = B // 128
L = 16
NC, NS = 2, 16
NW = NC * NS
CPW = B // NW
GROUPS = CPW // L

_mesh = plsc.VectorSubcoreMesh(core_axis_name="c", subcore_axis_name="s")


@functools.partial(
    pl.kernel,
    out_type=jax.ShapeDtypeStruct((RT, CT, 8, 128), jnp.float32),
    mesh=_mesh,
    scratch_types=[
        pltpu.VMEM((S, CPW), jnp.int32),
        pltpu.VMEM((RT, 8, CPW), jnp.float32),
        pltpu.SemaphoreType.DMA,
    ],
    compiler_params=pltpu.CompilerParams(
        use_tc_tiling_on_sc=False, needs_layout_passes=False
    ),
)
def _bow_kernel(idx_hbm, out_hbm, idx_v, hist_v, in_sem):
    wid = lax.axis_index("s") * NC + lax.axis_index("c")
    base = wid * CPW
    ct = base // 128
    ci0 = base % 128

    in_dma = pltpu.async_copy(idx_hbm.at[:, pl.ds(base, CPW)], idx_v, in_sem)

    zeros = jnp.zeros((L,), jnp.float32)

    @plsc.parallel_loop(0, RT * 8, step=1, unroll=8)
    def _zero(r):
        rt = r >> 3
        ri = r & 7
        hist_v[rt, ri, pl.ds(0, L)] = zeros
        hist_v[rt, ri, pl.ds(L, L)] = zeros

    in_dma.wait()

    ones = jnp.ones((L,), jnp.float32)
    lanes = lax.iota(jnp.int32, L)
    for g in range(GROUPS):
        cols = lanes + g * L

        @plsc.parallel_loop(0, S, step=1, unroll=5)
        def _step(t):
            tok = idx_v[t, pl.ds(g * L, L)]
            r = jnp.maximum(tok - 1, 0)
            plsc.addupdate_scatter(
                hist_v, [r >> 3, r & 7, cols], ones, mask=tok > 0
            )

    pltpu.sync_copy(hist_v, out_hbm.at[:, ct, :, pl.ds(ci0, CPW)])


def kernel(inputs):
    out4 = _bow_kernel(inputs.T)
    x = out4.transpose(0, 2, 1, 3).reshape(RT * 8, B)
    return x.T[:, :OUT_W]
